# Initial kernel scaffold; baseline (speedup 1.0000x reference)
import jax
import jax.numpy as jnp
from jax.experimental import pallas as pl


def kernel(x, batch, W1, b1, W2, b2):
    raise NotImplementedError("write your pallas kernel here")


# speedup vs baseline: 4.7381x; 4.7381x over previous
import jax
import jax.numpy as jnp
from jax import lax
from jax.experimental import pallas as pl
from jax.experimental.pallas import tpu as pltpu

N = 100000
HIDDEN = 128
ATTN = 128
B = 512
BLK = 512
NBLK = (N + BLK - 1) // BLK
NPAD = NBLK * BLK

NEG = -1e30


def _k1_body(x_ref, b3_ref, w1_ref, b1_ref, w2_ref, s_ref, m_ref):
    pid = pl.program_id(0)
    x = x_ref[...]
    h = jnp.tanh(
        jax.lax.dot_general(x, w1_ref[...], (((1,), (0,)), ((), ())),
                            preferred_element_type=jnp.float32)
        + b1_ref[...]
    )
    s_row = jax.lax.dot_general(w2_ref[...], h, (((1,), (1,)), ((), ())),
                                preferred_element_type=jnp.float32)
    s_ref[...] = s_row[None]

    b_row = b3_ref[0, 0, :].reshape(1, BLK)
    gid = pid * BLK + lax.broadcasted_iota(jnp.int32, (1, BLK), 1)
    valid = gid < N
    seg_col = lax.broadcasted_iota(jnp.int32, (B, 1), 0)
    onehot = (seg_col == b_row) & valid
    masked = jnp.where(onehot, s_row, NEG)
    m_blk = jnp.max(masked, axis=1, keepdims=True)

    @pl.when(pid == 0)
    def _():
        m_ref[...] = jnp.full((B, 1), NEG, jnp.float32)

    m_ref[...] = jnp.maximum(m_ref[...], m_blk)


def _k2_body(x_ref, s_ref, b3_ref, m_ref, out_ref, denom_ref):
    pid = pl.program_id(0)
    s_row = s_ref[0, :, :]
    b_row = b3_ref[0, 0, :].reshape(1, BLK)
    gid = pid * BLK + lax.broadcasted_iota(jnp.int32, (1, BLK), 1)
    valid = gid < N
    seg_col = lax.broadcasted_iota(jnp.int32, (B, 1), 0)
    onehot = (seg_col == b_row) & valid

    m_r = jnp.sum(jnp.where(onehot, m_ref[...], 0.0), axis=0, keepdims=True)
    e_row = jnp.where(valid, jnp.exp(s_row - m_r), 0.0)
    a = jnp.where(onehot, e_row, 0.0)

    @pl.when(pid == 0)
    def _():
        denom_ref[...] = jnp.zeros((B, 1), jnp.float32)
        out_ref[...] = jnp.zeros((B, HIDDEN), jnp.float32)

    denom_ref[...] += jnp.sum(a, axis=1, keepdims=True)
    out_ref[...] += jax.lax.dot_general(a, x_ref[...], (((1,), (0,)), ((), ())),
                                        preferred_element_type=jnp.float32)

    @pl.when(pid == pl.num_programs(0) - 1)
    def _():
        out_ref[...] = out_ref[...] / (denom_ref[...] + 1e-16)


def kernel(x, batch, W1, b1, W2, b2):
    del b2
    xp = jnp.pad(x, ((0, NPAD - N), (0, 0)))
    b3 = jnp.pad(batch.astype(jnp.int32), (0, NPAD - N)).reshape(NBLK, 1, BLK)
    b1r = b1.reshape(1, HIDDEN)
    w2r = W2.reshape(1, ATTN)

    s3, m = pl.pallas_call(
        _k1_body,
        grid=(NBLK,),
        in_specs=[
            pl.BlockSpec((BLK, HIDDEN), lambda i: (i, 0)),
            pl.BlockSpec((1, 1, BLK), lambda i: (i, 0, 0)),
            pl.BlockSpec((HIDDEN, ATTN), lambda i: (0, 0)),
            pl.BlockSpec((1, ATTN), lambda i: (0, 0)),
            pl.BlockSpec((1, ATTN), lambda i: (0, 0)),
        ],
        out_specs=[
            pl.BlockSpec((1, 1, BLK), lambda i: (i, 0, 0)),
            pl.BlockSpec((B, 1), lambda i: (0, 0)),
        ],
        out_shape=[
            jax.ShapeDtypeStruct((NBLK, 1, BLK), jnp.float32),
            jax.ShapeDtypeStruct((B, 1), jnp.float32),
        ],
    )(xp, b3, W1, b1r, w2r)

    pooled = pl.pallas_call(
        _k2_body,
        grid=(NBLK,),
        in_specs=[
            pl.BlockSpec((BLK, HIDDEN), lambda i: (i, 0)),
            pl.BlockSpec((1, 1, BLK), lambda i: (i, 0, 0)),
            pl.BlockSpec((1, 1, BLK), lambda i: (i, 0, 0)),
            pl.BlockSpec((B, 1), lambda i: (0, 0)),
        ],
        out_specs=[
            pl.BlockSpec((B, HIDDEN), lambda i: (0, 0)),
            pl.BlockSpec((B, 1), lambda i: (0, 0)),
        ],
        out_shape=[
            jax.ShapeDtypeStruct((B, HIDDEN), jnp.float32),
            jax.ShapeDtypeStruct((B, 1), jnp.float32),
        ],
    )(xp, s3, b3, m)[0]

    return pooled


# speedup vs baseline: 8.0078x; 1.6901x over previous
import jax
import jax.numpy as jnp
from jax import lax
from jax.experimental import pallas as pl
from jax.experimental.pallas import tpu as pltpu

N = 100000
HIDDEN = 128
ATTN = 128
B = 512
BLK = 512
NBLK = (N + BLK - 1) // BLK
NPAD = NBLK * BLK
SW = 64

NEG = -1e30


def _body(base_ref, width_ref, x_ref, b3_ref, w1_ref, b1_ref, w2_ref,
          out_ref, m_scr, d_scr):
    pid = pl.program_id(0)

    @pl.when(pid == 0)
    def _():
        m_scr[...] = jnp.full((B, 1), NEG, jnp.float32)
        d_scr[...] = jnp.zeros((B, 1), jnp.float32)
        out_ref[...] = jnp.zeros((B, HIDDEN), jnp.float32)

    x = x_ref[...]
    h = jnp.tanh(
        lax.dot_general(x, w1_ref[...], (((1,), (0,)), ((), ())),
                        preferred_element_type=jnp.float32)
        + b1_ref[...]
    )
    s_row = lax.dot_general(w2_ref[...], h, (((1,), (1,)), ((), ())),
                            preferred_element_type=jnp.float32)

    b_row = b3_ref[0, 0, :].reshape(1, BLK)
    gid = pid * BLK + lax.broadcasted_iota(jnp.int32, (1, BLK), 1)
    valid = gid < N
    base = base_ref[pid]
    width = width_ref[pid]

    def online_update(seg_col, m_old, d_old, o_old):
        onehot = (seg_col == b_row) & valid
        m_blk = jnp.max(jnp.where(onehot, s_row, NEG), axis=1, keepdims=True)
        m_new = jnp.maximum(m_old, m_blk)
        alpha = jnp.exp(m_old - m_new)
        m_r = jnp.sum(jnp.where(onehot, m_new, 0.0), axis=0, keepdims=True)
        e = jnp.where(valid, jnp.exp(s_row - m_r), 0.0)
        a = jnp.where(onehot, e, 0.0)
        d_new = d_old * alpha + jnp.sum(a, axis=1, keepdims=True)
        o_new = o_old * alpha + lax.dot_general(
            a, x, (((1,), (0,)), ((), ())), preferred_element_type=jnp.float32)
        return m_new, d_new, o_new

    @pl.when(width <= SW)
    def _():
        cbase = jnp.minimum(base, B - SW)
        sl = pl.ds(cbase, SW)
        seg_col = cbase + lax.broadcasted_iota(jnp.int32, (SW, 1), 0)
        m_new, d_new, o_new = online_update(
            seg_col, m_scr[sl, :], d_scr[sl, :], out_ref[sl, :])
        m_scr[sl, :] = m_new
        d_scr[sl, :] = d_new
        out_ref[sl, :] = o_new

    @pl.when(width > SW)
    def _():
        seg_col = lax.broadcasted_iota(jnp.int32, (B, 1), 0)
        m_new, d_new, o_new = online_update(
            seg_col, m_scr[...], d_scr[...], out_ref[...])
        m_scr[...] = m_new
        d_scr[...] = d_new
        out_ref[...] = o_new

    @pl.when(pid == pl.num_programs(0) - 1)
    def _():
        out_ref[...] = out_ref[...] / (d_scr[...] + 1e-16)


def kernel(x, batch, W1, b1, W2, b2):
    del b2
    bi = batch.astype(jnp.int32)
    xp = jnp.pad(x, ((0, NPAD - N), (0, 0)))
    b3 = jnp.pad(bi, (0, NPAD - N), constant_values=B - 1).reshape(NBLK, 1, BLK)
    idx = jnp.arange(NBLK, dtype=jnp.int32)
    bases = bi[jnp.minimum(idx * BLK, N - 1)]
    lasts = bi[jnp.minimum((idx + 1) * BLK - 1, N - 1)]
    widths = lasts - bases + 1
    b1r = b1.reshape(1, HIDDEN)
    w2r = W2.reshape(1, ATTN)

    pooled = pl.pallas_call(
        _body,
        grid=(NBLK,),
        in_specs=[
            pl.BlockSpec(memory_space=pltpu.SMEM),
            pl.BlockSpec(memory_space=pltpu.SMEM),
            pl.BlockSpec((BLK, HIDDEN), lambda i: (i, 0)),
            pl.BlockSpec((1, 1, BLK), lambda i: (i, 0, 0)),
            pl.BlockSpec((HIDDEN, ATTN), lambda i: (0, 0)),
            pl.BlockSpec((1, ATTN), lambda i: (0, 0)),
            pl.BlockSpec((1, ATTN), lambda i: (0, 0)),
        ],
        out_specs=pl.BlockSpec((B, HIDDEN), lambda i: (0, 0)),
        out_shape=jax.ShapeDtypeStruct((B, HIDDEN), jnp.float32),
        scratch_shapes=[
            pltpu.VMEM((B, 1), jnp.float32),
            pltpu.VMEM((B, 1), jnp.float32),
        ],
    )(bases, widths, xp, b3, W1, b1r, w2r)

    return pooled


# speedup vs baseline: 12.7508x; 1.5923x over previous
import jax
import jax.numpy as jnp
from jax import lax
from jax.experimental import pallas as pl
from jax.experimental.pallas import tpu as pltpu

N = 100000
HIDDEN = 128
ATTN = 128
B = 512
BLK = 1024
NBLK = (N + BLK - 1) // BLK
NPAD = NBLK * BLK
SW = 16

NEG = -1e30


def _body(base_ref, width_ref, x_ref, b3_ref, w1_ref, b1_ref, w2_ref,
          out_ref, m_scr, d_scr):
    pid = pl.program_id(0)

    @pl.when(pid == 0)
    def _():
        m_scr[...] = jnp.full((B, 1), NEG, jnp.float32)
        d_scr[...] = jnp.zeros((B, 1), jnp.float32)
        out_ref[...] = jnp.zeros((B, HIDDEN), jnp.float32)

    x = x_ref[...]
    h = jnp.tanh(
        lax.dot_general(x, w1_ref[...], (((1,), (0,)), ((), ())),
                        preferred_element_type=jnp.float32)
        + b1_ref[...]
    )
    s_row = lax.dot_general(w2_ref[...], h, (((1,), (1,)), ((), ())),
                            preferred_element_type=jnp.float32)

    b_row = b3_ref[0, 0, :].reshape(1, BLK)
    gid = pid * BLK + lax.broadcasted_iota(jnp.int32, (1, BLK), 1)
    valid = gid < N
    base = base_ref[pid]
    width = width_ref[pid]

    def online_update(seg_col, m_old, d_old, o_old):
        onehot = (seg_col == b_row) & valid
        m_blk = jnp.max(jnp.where(onehot, s_row, NEG), axis=1, keepdims=True)
        m_new = jnp.maximum(m_old, m_blk)
        alpha = jnp.exp(m_old - m_new)
        m_r = jnp.sum(jnp.where(onehot, m_new, 0.0), axis=0, keepdims=True)
        e = jnp.where(valid, jnp.exp(s_row - m_r), 0.0)
        a = jnp.where(onehot, e, 0.0)
        d_new = d_old * alpha + jnp.sum(a, axis=1, keepdims=True)
        o_new = o_old * alpha + lax.dot_general(
            a, x, (((1,), (0,)), ((), ())), preferred_element_type=jnp.float32)
        return m_new, d_new, o_new

    @pl.when(width <= SW)
    def _():
        cbase = jnp.minimum(base, B - SW)
        sl = pl.ds(cbase, SW)
        seg_col = cbase + lax.broadcasted_iota(jnp.int32, (SW, 1), 0)
        m_new, d_new, o_new = online_update(
            seg_col, m_scr[sl, :], d_scr[sl, :], out_ref[sl, :])
        m_scr[sl, :] = m_new
        d_scr[sl, :] = d_new
        out_ref[sl, :] = o_new

    @pl.when(width > SW)
    def _():
        seg_col = lax.broadcasted_iota(jnp.int32, (B, 1), 0)
        m_new, d_new, o_new = online_update(
            seg_col, m_scr[...], d_scr[...], out_ref[...])
        m_scr[...] = m_new
        d_scr[...] = d_new
        out_ref[...] = o_new

    @pl.when(pid == pl.num_programs(0) - 1)
    def _():
        out_ref[...] = out_ref[...] / (d_scr[...] + 1e-16)


def kernel(x, batch, W1, b1, W2, b2):
    del b2
    bi = batch.astype(jnp.int32)
    xp = jnp.pad(x, ((0, NPAD - N), (0, 0)))
    b3 = jnp.pad(bi, (0, NPAD - N), constant_values=B - 1).reshape(NBLK, 1, BLK)
    idx = jnp.arange(NBLK, dtype=jnp.int32)
    bases = bi[jnp.minimum(idx * BLK, N - 1)]
    lasts = bi[jnp.minimum((idx + 1) * BLK - 1, N - 1)]
    widths = lasts - bases + 1
    b1r = b1.reshape(1, HIDDEN)
    w2r = W2.reshape(1, ATTN)

    pooled = pl.pallas_call(
        _body,
        grid=(NBLK,),
        in_specs=[
            pl.BlockSpec(memory_space=pltpu.SMEM),
            pl.BlockSpec(memory_space=pltpu.SMEM),
            pl.BlockSpec((BLK, HIDDEN), lambda i: (i, 0)),
            pl.BlockSpec((1, 1, BLK), lambda i: (i, 0, 0)),
            pl.BlockSpec((HIDDEN, ATTN), lambda i: (0, 0)),
            pl.BlockSpec((1, ATTN), lambda i: (0, 0)),
            pl.BlockSpec((1, ATTN), lambda i: (0, 0)),
        ],
        out_specs=pl.BlockSpec((B, HIDDEN), lambda i: (0, 0)),
        out_shape=jax.ShapeDtypeStruct((B, HIDDEN), jnp.float32),
        scratch_shapes=[
            pltpu.VMEM((B, 1), jnp.float32),
            pltpu.VMEM((B, 1), jnp.float32),
        ],
    )(bases, widths, xp, b3, W1, b1r, w2r)

    return pooled


# speedup vs baseline: 13.0098x; 1.0203x over previous
import jax
import jax.numpy as jnp
from jax import lax
from jax.experimental import pallas as pl
from jax.experimental.pallas import tpu as pltpu

N = 100000
HIDDEN = 128
ATTN = 128
B = 512
BLK = 1024
NBLK = (N + BLK - 1) // BLK
NPAD = NBLK * BLK
SW = 16

NEG = -1e30


def _body(base_ref, width_ref, x_ref, b3_ref, w1_ref, b1_ref, w2_ref,
          out_ref, m_scr, d_scr):
    pid = pl.program_id(0)

    @pl.when(pid == 0)
    def _():
        m_scr[0, 0] = NEG
        d_scr[...] = jnp.zeros((B, 1), jnp.float32)
        out_ref[...] = jnp.zeros((B, HIDDEN), jnp.float32)

    x = x_ref[...]
    h = jnp.tanh(
        lax.dot_general(x, w1_ref[...], (((1,), (0,)), ((), ())),
                        preferred_element_type=jnp.float32)
        + b1_ref[...]
    )
    s_row = lax.dot_general(w2_ref[...], h, (((1,), (1,)), ((), ())),
                            preferred_element_type=jnp.float32)

    m_old = m_scr[0, 0]
    m_new = jnp.maximum(m_old, jnp.max(s_row))

    @pl.when(m_new > m_old)
    def _():
        scale = jnp.exp(m_old - m_new)
        d_scr[...] = d_scr[...] * scale
        out_ref[...] = out_ref[...] * scale
        m_scr[0, 0] = m_new

    b_row = b3_ref[0, 0, :].reshape(1, BLK)
    gid = pid * BLK + lax.broadcasted_iota(jnp.int32, (1, BLK), 1)
    e = jnp.where(gid < N, jnp.exp(s_row - m_new), 0.0)
    base = base_ref[pid]
    width = width_ref[pid]

    def accumulate(seg_col, d_ref_sl, o_ref_sl):
        a = jnp.where(seg_col == b_row, e, 0.0)
        d_ref_sl[...] += jnp.sum(a, axis=1, keepdims=True)
        o_ref_sl[...] += lax.dot_general(
            a, x, (((1,), (0,)), ((), ())), preferred_element_type=jnp.float32)

    @pl.when(width <= SW)
    def _():
        cbase = jnp.minimum(base, B - SW)
        seg_col = cbase + lax.broadcasted_iota(jnp.int32, (SW, 1), 0)
        accumulate(seg_col, d_scr.at[pl.ds(cbase, SW), :],
                   out_ref.at[pl.ds(cbase, SW), :])

    @pl.when(width > SW)
    def _():
        seg_col = lax.broadcasted_iota(jnp.int32, (B, 1), 0)
        accumulate(seg_col, d_scr.at[...], out_ref.at[...])

    @pl.when(pid == pl.num_programs(0) - 1)
    def _():
        out_ref[...] = out_ref[...] / (d_scr[...] + 1e-16)


def kernel(x, batch, W1, b1, W2, b2):
    del b2
    bi = batch.astype(jnp.int32)
    xp = jnp.pad(x, ((0, NPAD - N), (0, 0)))
    b3 = jnp.pad(bi, (0, NPAD - N), constant_values=B - 1).reshape(NBLK, 1, BLK)
    idx = jnp.arange(NBLK, dtype=jnp.int32)
    bases = bi[jnp.minimum(idx * BLK, N - 1)]
    lasts = bi[jnp.minimum((idx + 1) * BLK - 1, N - 1)]
    widths = lasts - bases + 1
    b1r = b1.reshape(1, HIDDEN)
    w2r = W2.reshape(1, ATTN)

    pooled = pl.pallas_call(
        _body,
        grid=(NBLK,),
        in_specs=[
            pl.BlockSpec(memory_space=pltpu.SMEM),
            pl.BlockSpec(memory_space=pltpu.SMEM),
            pl.BlockSpec((BLK, HIDDEN), lambda i: (i, 0)),
            pl.BlockSpec((1, 1, BLK), lambda i: (i, 0, 0)),
            pl.BlockSpec((HIDDEN, ATTN), lambda i: (0, 0)),
            pl.BlockSpec((1, ATTN), lambda i: (0, 0)),
            pl.BlockSpec((1, ATTN), lambda i: (0, 0)),
        ],
        out_specs=pl.BlockSpec((B, HIDDEN), lambda i: (0, 0)),
        out_shape=jax.ShapeDtypeStruct((B, HIDDEN), jnp.float32),
        scratch_shapes=[
            pltpu.SMEM((1, 1), jnp.float32),
            pltpu.VMEM((B, 1), jnp.float32),
        ],
    )(bases, widths, xp, b3, W1, b1r, w2r)

    return pooled


# speedup vs baseline: 18.0464x; 1.3871x over previous
import jax
import jax.numpy as jnp
from jax import lax
from jax.experimental import pallas as pl
from jax.experimental.pallas import tpu as pltpu

N = 100000
HIDDEN = 128
ATTN = 128
B = 512
BLK = 2048
NBLK = (N + BLK - 1) // BLK
NPAD = NBLK * BLK
SW = 16

NEG = -1e30


def _body(base_ref, width_ref, x_ref, b3_ref, w1_ref, b1_ref, w2_ref,
          out_ref, m_scr, d_scr):
    pid = pl.program_id(0)

    @pl.when(pid == 0)
    def _():
        m_scr[0, 0] = NEG
        d_scr[...] = jnp.zeros((B, 1), jnp.float32)
        out_ref[...] = jnp.zeros((B, HIDDEN), jnp.float32)

    x = x_ref[...]
    h = jnp.tanh(
        lax.dot_general(x.astype(jnp.bfloat16), w1_ref[...].astype(jnp.bfloat16),
                        (((1,), (0,)), ((), ())),
                        preferred_element_type=jnp.float32)
        + b1_ref[...]
    )
    s_row = lax.dot_general(w2_ref[...].astype(jnp.bfloat16),
                            h.astype(jnp.bfloat16), (((1,), (1,)), ((), ())),
                            preferred_element_type=jnp.float32)

    m_old = m_scr[0, 0]
    m_new = jnp.maximum(m_old, jnp.max(s_row))

    @pl.when(m_new > m_old)
    def _():
        scale = jnp.exp(m_old - m_new)
        d_scr[...] = d_scr[...] * scale
        out_ref[...] = out_ref[...] * scale
        m_scr[0, 0] = m_new

    b_row = b3_ref[0, 0, :].reshape(1, BLK)
    gid = pid * BLK + lax.broadcasted_iota(jnp.int32, (1, BLK), 1)
    e = jnp.where(gid < N, jnp.exp(s_row - m_new), 0.0)
    base = base_ref[pid]
    width = width_ref[pid]

    def accumulate(seg_col, d_ref_sl, o_ref_sl):
        a = jnp.where(seg_col == b_row, e, 0.0)
        d_ref_sl[...] += jnp.sum(a, axis=1, keepdims=True)
        o_ref_sl[...] += lax.dot_general(
            a, x, (((1,), (0,)), ((), ())), preferred_element_type=jnp.float32)

    @pl.when(width <= SW)
    def _():
        cbase = jnp.minimum(base, B - SW)
        seg_col = cbase + lax.broadcasted_iota(jnp.int32, (SW, 1), 0)
        accumulate(seg_col, d_scr.at[pl.ds(cbase, SW), :],
                   out_ref.at[pl.ds(cbase, SW), :])

    @pl.when(width > SW)
    def _():
        seg_col = lax.broadcasted_iota(jnp.int32, (B, 1), 0)
        accumulate(seg_col, d_scr.at[...], out_ref.at[...])

    @pl.when(pid == pl.num_programs(0) - 1)
    def _():
        out_ref[...] = out_ref[...] / (d_scr[...] + 1e-16)


def kernel(x, batch, W1, b1, W2, b2):
    del b2
    bi = batch.astype(jnp.int32)
    xp = jnp.pad(x, ((0, NPAD - N), (0, 0)))
    b3 = jnp.pad(bi, (0, NPAD - N), constant_values=B - 1).reshape(NBLK, 1, BLK)
    idx = jnp.arange(NBLK, dtype=jnp.int32)
    bases = bi[jnp.minimum(idx * BLK, N - 1)]
    lasts = bi[jnp.minimum((idx + 1) * BLK - 1, N - 1)]
    widths = lasts - bases + 1
    b1r = b1.reshape(1, HIDDEN)
    w2r = W2.reshape(1, ATTN)

    pooled = pl.pallas_call(
        _body,
        grid=(NBLK,),
        in_specs=[
            pl.BlockSpec(memory_space=pltpu.SMEM),
            pl.BlockSpec(memory_space=pltpu.SMEM),
            pl.BlockSpec((BLK, HIDDEN), lambda i: (i, 0)),
            pl.BlockSpec((1, 1, BLK), lambda i: (i, 0, 0)),
            pl.BlockSpec((HIDDEN, ATTN), lambda i: (0, 0)),
            pl.BlockSpec((1, ATTN), lambda i: (0, 0)),
            pl.BlockSpec((1, ATTN), lambda i: (0, 0)),
        ],
        out_specs=pl.BlockSpec((B, HIDDEN), lambda i: (0, 0)),
        out_shape=jax.ShapeDtypeStruct((B, HIDDEN), jnp.float32),
        scratch_shapes=[
            pltpu.SMEM((1, 1), jnp.float32),
            pltpu.VMEM((B, 1), jnp.float32),
        ],
    )(bases, widths, xp, b3, W1, b1r, w2r)

    return pooled


# speedup vs baseline: 20.8524x; 1.1555x over previous
import jax
import jax.numpy as jnp
from jax import lax
from jax.experimental import pallas as pl
from jax.experimental.pallas import tpu as pltpu

N = 100000
HIDDEN = 128
ATTN = 128
B = 512
SUB = 1024
NSUBBLK = 4
BLK = SUB * NSUBBLK
NBLK = (N + BLK - 1) // BLK
NSUB = NBLK * NSUBBLK
SW = 16

NEG = -1e30
TAILV = N - (NBLK - 1) * BLK


def _body(base_ref, width_ref, x_ref, b3_ref, w1_ref, b1_ref, w2_ref,
          out_ref, m_scr, d_scr):
    pid = pl.program_id(0)

    @pl.when(pid == 0)
    def _():
        m_scr[0, 0] = NEG
        d_scr[...] = jnp.zeros((B, 1), jnp.float32)
        out_ref[...] = jnp.zeros((B, HIDDEN), jnp.float32)

    @pl.when(pid == NBLK - 1)
    def _():
        x_ref[TAILV:, :] = jnp.zeros((BLK - TAILV, HIDDEN), jnp.float32)

    w1 = w1_ref[...].astype(jnp.bfloat16)
    w2 = w2_ref[...].astype(jnp.bfloat16)
    b1 = b1_ref[...]

    for j in range(NSUBBLK):
        x = x_ref[j * SUB:(j + 1) * SUB, :]
        h = jnp.tanh(
            lax.dot_general(x.astype(jnp.bfloat16), w1, (((1,), (0,)), ((), ())),
                            preferred_element_type=jnp.float32)
            + b1
        )
        s_row = lax.dot_general(w2, h.astype(jnp.bfloat16),
                                (((1,), (1,)), ((), ())),
                                preferred_element_type=jnp.float32)

        gid = pid * BLK + j * SUB + lax.broadcasted_iota(jnp.int32, (1, SUB), 1)
        valid = gid < N
        s_row = jnp.where(valid, s_row, NEG)

        m_old = m_scr[0, 0]
        m_new = jnp.maximum(m_old, jnp.max(s_row))

        @pl.when(m_new > m_old)
        def _():
            scale = jnp.exp(m_old - m_new)
            d_scr[...] = d_scr[...] * scale
            out_ref[...] = out_ref[...] * scale
            m_scr[0, 0] = m_new

        b_row = b3_ref[0, j, :].reshape(1, SUB)
        e = jnp.where(valid, jnp.exp(s_row - m_new), 0.0)
        sp = pid * NSUBBLK + j
        base = base_ref[sp]
        width = width_ref[sp]

        def accumulate(seg_col, d_ref_sl, o_ref_sl, a_e=e, a_x=x):
            a = jnp.where(seg_col == b_row, a_e, 0.0)
            d_ref_sl[...] += jnp.sum(a, axis=1, keepdims=True)
            o_ref_sl[...] += lax.dot_general(
                a, a_x, (((1,), (0,)), ((), ())),
                preferred_element_type=jnp.float32)

        @pl.when(width <= SW)
        def _():
            cbase = jnp.minimum(base, B - SW)
            seg_col = cbase + lax.broadcasted_iota(jnp.int32, (SW, 1), 0)
            accumulate(seg_col, d_scr.at[pl.ds(cbase, SW), :],
                       out_ref.at[pl.ds(cbase, SW), :])

        @pl.when(width > SW)
        def _():
            seg_col = lax.broadcasted_iota(jnp.int32, (B, 1), 0)
            accumulate(seg_col, d_scr.at[...], out_ref.at[...])

    @pl.when(pid == pl.num_programs(0) - 1)
    def _():
        out_ref[...] = out_ref[...] / (d_scr[...] + 1e-16)


def kernel(x, batch, W1, b1, W2, b2):
    del b2
    bi = batch.astype(jnp.int32)
    b3 = jnp.pad(bi, (0, NBLK * BLK - N), constant_values=B - 1).reshape(
        NBLK, NSUBBLK, SUB)
    idx = jnp.arange(NSUB, dtype=jnp.int32)
    bases = bi[jnp.minimum(idx * SUB, N - 1)]
    lasts = bi[jnp.minimum((idx + 1) * SUB - 1, N - 1)]
    widths = lasts - bases + 1
    b1r = b1.reshape(1, HIDDEN)
    w2r = W2.reshape(1, ATTN)

    pooled = pl.pallas_call(
        _body,
        grid=(NBLK,),
        in_specs=[
            pl.BlockSpec(memory_space=pltpu.SMEM),
            pl.BlockSpec(memory_space=pltpu.SMEM),
            pl.BlockSpec((BLK, HIDDEN), lambda i: (i, 0)),
            pl.BlockSpec((1, NSUBBLK, SUB), lambda i: (i, 0, 0)),
            pl.BlockSpec((HIDDEN, ATTN), lambda i: (0, 0)),
            pl.BlockSpec((1, ATTN), lambda i: (0, 0)),
            pl.BlockSpec((1, ATTN), lambda i: (0, 0)),
        ],
        out_specs=pl.BlockSpec((B, HIDDEN), lambda i: (0, 0)),
        out_shape=jax.ShapeDtypeStruct((B, HIDDEN), jnp.float32),
        scratch_shapes=[
            pltpu.SMEM((1, 1), jnp.float32),
            pltpu.VMEM((B, 1), jnp.float32),
        ],
    )(bases, widths, x, b3, W1, b1r, w2r)

    return pooled


# speedup vs baseline: 24.7544x; 1.1871x over previous
import jax
import jax.numpy as jnp
from jax import lax
from jax.experimental import pallas as pl
from jax.experimental.pallas import tpu as pltpu

N = 100000
HIDDEN = 128
ATTN = 128
B = 512
SUB = 1024
NSUBBLK = 4
BLK = SUB * NSUBBLK
NBLK = (N + BLK - 1) // BLK
NSUB = NBLK * NSUBBLK
SW = 16

NEG = -1e30
TAILV = N - (NBLK - 1) * BLK


def _body(base_ref, width_ref, x_ref, b3_ref, w1_ref, b1_ref, w2_ref,
          out_ref, d_scr):
    pid = pl.program_id(0)

    @pl.when(pid == 0)
    def _():
        d_scr[...] = jnp.zeros((B, 1), jnp.float32)
        out_ref[...] = jnp.zeros((B, HIDDEN), jnp.float32)

    @pl.when(pid == NBLK - 1)
    def _():
        x_ref[TAILV:, :] = jnp.zeros((BLK - TAILV, HIDDEN), jnp.float32)

    w1 = w1_ref[...].astype(jnp.bfloat16)
    w2f = w2_ref[...]
    w2 = w2f.astype(jnp.bfloat16)
    b1 = b1_ref[...]
    m0 = jnp.sum(jnp.abs(w2f))

    for j in range(NSUBBLK):
        x = x_ref[j * SUB:(j + 1) * SUB, :]
        h = jnp.tanh(
            lax.dot_general(x.astype(jnp.bfloat16), w1, (((1,), (0,)), ((), ())),
                            preferred_element_type=jnp.float32)
            + b1
        )
        s_row = lax.dot_general(w2, h.astype(jnp.bfloat16),
                                (((1,), (1,)), ((), ())),
                                preferred_element_type=jnp.float32)

        gid = pid * BLK + j * SUB + lax.broadcasted_iota(jnp.int32, (1, SUB), 1)
        valid = gid < N

        b_row = b3_ref[0, j, :].reshape(1, SUB)
        e = jnp.where(valid, jnp.exp(s_row - m0), 0.0)
        sp = pid * NSUBBLK + j
        base = base_ref[sp]
        width = width_ref[sp]

        def accumulate(seg_col, d_ref_sl, o_ref_sl, a_e=e, a_x=x):
            a = jnp.where(seg_col == b_row, a_e, 0.0)
            d_ref_sl[...] += jnp.sum(a, axis=1, keepdims=True)
            o_ref_sl[...] += lax.dot_general(
                a, a_x, (((1,), (0,)), ((), ())),
                preferred_element_type=jnp.float32)

        @pl.when(width <= SW)
        def _():
            cbase = jnp.minimum(base, B - SW)
            seg_col = cbase + lax.broadcasted_iota(jnp.int32, (SW, 1), 0)
            accumulate(seg_col, d_scr.at[pl.ds(cbase, SW), :],
                       out_ref.at[pl.ds(cbase, SW), :])

        @pl.when(width > SW)
        def _():
            seg_col = lax.broadcasted_iota(jnp.int32, (B, 1), 0)
            accumulate(seg_col, d_scr.at[...], out_ref.at[...])

    @pl.when(pid == pl.num_programs(0) - 1)
    def _():
        out_ref[...] = out_ref[...] / (d_scr[...] + 1e-16)


def kernel(x, batch, W1, b1, W2, b2):
    del b2
    bi = batch.astype(jnp.int32)
    b3 = jnp.pad(bi, (0, NBLK * BLK - N), constant_values=B - 1).reshape(
        NBLK, NSUBBLK, SUB)
    idx = jnp.arange(NSUB, dtype=jnp.int32)
    bases = bi[jnp.minimum(idx * SUB, N - 1)]
    lasts = bi[jnp.minimum((idx + 1) * SUB - 1, N - 1)]
    widths = lasts - bases + 1
    b1r = b1.reshape(1, HIDDEN)
    w2r = W2.reshape(1, ATTN)

    pooled = pl.pallas_call(
        _body,
        grid=(NBLK,),
        in_specs=[
            pl.BlockSpec(memory_space=pltpu.SMEM),
            pl.BlockSpec(memory_space=pltpu.SMEM),
            pl.BlockSpec((BLK, HIDDEN), lambda i: (i, 0)),
            pl.BlockSpec((1, NSUBBLK, SUB), lambda i: (i, 0, 0)),
            pl.BlockSpec((HIDDEN, ATTN), lambda i: (0, 0)),
            pl.BlockSpec((1, ATTN), lambda i: (0, 0)),
            pl.BlockSpec((1, ATTN), lambda i: (0, 0)),
        ],
        out_specs=pl.BlockSpec((B, HIDDEN), lambda i: (0, 0)),
        out_shape=jax.ShapeDtypeStruct((B, HIDDEN), jnp.float32),
        scratch_shapes=[
            pltpu.VMEM((B, 1), jnp.float32),
        ],
    )(bases, widths, x, b3, W1, b1r, w2r)

    return pooled
